# baseline (speedup 1.0000x reference)
import functools

import jax
import jax.numpy as jnp
from jax import lax
from jax.experimental import pallas as pl
from jax.experimental.pallas import tpu as pltpu
from jax.experimental.pallas import tpu_sc as plsc

NC, NS = 2, 16
NW = NC * NS
B = 16384
BW = B // NW
C = 128
NCHUNK = BW // C
DG = 64
DM = 256


def _sc_gather_body(user_hbm, item_hbm, ug_tab, ig_tab, um_tab, im_tab,
                    ug_out, ig_out, um_out, im_out,
                    idx_u, idx_i, ug_v, ig_v, um_v, im_v,
                    s1, s2, s3, s4):
    wid = lax.axis_index("s") * NC + lax.axis_index("c")
    for c in range(NCHUNK):
        base = wid * BW + c * C
        pltpu.sync_copy(user_hbm.at[pl.ds(base, C)], idx_u)
        pltpu.sync_copy(item_hbm.at[pl.ds(base, C)], idx_i)
        cp_um = pltpu.async_copy(um_tab.at[idx_u], um_v, s3)
        cp_im = pltpu.async_copy(im_tab.at[idx_i], im_v, s4)
        cp_ug = pltpu.async_copy(ug_tab.at[idx_u], ug_v, s1)
        cp_ig = pltpu.async_copy(ig_tab.at[idx_i], ig_v, s2)
        cp_ug.wait()
        pltpu.sync_copy(ug_v, ug_out.at[pl.ds(base, C)])
        cp_ig.wait()
        pltpu.sync_copy(ig_v, ig_out.at[pl.ds(base, C)])
        cp_um.wait()
        pltpu.sync_copy(um_v, um_out.at[pl.ds(base, C)])
        cp_im.wait()
        pltpu.sync_copy(im_v, im_out.at[pl.ds(base, C)])


def _sc_gather(user, item, ug_tab, ig_tab, um_tab, im_tab):
    mesh = plsc.VectorSubcoreMesh(core_axis_name="c", subcore_axis_name="s",
                                  num_cores=NC, num_subcores=NS)
    f32 = jnp.float32
    fn = pl.kernel(
        _sc_gather_body,
        compiler_params=pltpu.CompilerParams(use_tc_tiling_on_sc=False),
        out_type=[
            jax.ShapeDtypeStruct((B, DG), f32),
            jax.ShapeDtypeStruct((B, DG), f32),
            jax.ShapeDtypeStruct((B, DM), f32),
            jax.ShapeDtypeStruct((B, DM), f32),
        ],
        mesh=mesh,
        scratch_types=[
            pltpu.VMEM((C,), jnp.int32),
            pltpu.VMEM((C,), jnp.int32),
            pltpu.VMEM((C, DG), f32),
            pltpu.VMEM((C, DG), f32),
            pltpu.VMEM((C, DM), f32),
            pltpu.VMEM((C, DM), f32),
            pltpu.SemaphoreType.DMA,
            pltpu.SemaphoreType.DMA,
            pltpu.SemaphoreType.DMA,
            pltpu.SemaphoreType.DMA,
        ],
    )
    return fn(user, item, ug_tab, ig_tab, um_tab, im_tab)


TB = 2048


def _tc_mlp_body(ug, ig, um, im, w1a, w1b, b1, w2, b2, w3, b3, wpa, wpb, bp,
                 out):
    f32 = jnp.float32
    hp = jax.lax.Precision.HIGHEST
    h = jnp.dot(um[...], w1a[...], precision=hp, preferred_element_type=f32)
    h += jnp.dot(im[...], w1b[...], precision=hp, preferred_element_type=f32)
    h = jnp.maximum(h + b1[...], 0.0)
    h = jnp.maximum(
        jnp.dot(h, w2[...], precision=hp, preferred_element_type=f32)
        + b2[...], 0.0)
    h = jnp.maximum(
        jnp.dot(h, w3[...], precision=hp, preferred_element_type=f32)
        + b3[...], 0.0)
    gmf = ug[...] * ig[...]
    pred = (jnp.sum(gmf * wpa[...], axis=1) + jnp.sum(h * wpb[...], axis=1)
            + bp[0, 0])
    out[...] = pred


def _tc_mlp(ug, ig, um, im, W1, b1, W2, b2, W3, b3, Wp, bp):
    f32 = jnp.float32
    w1a, w1b = W1[:DM], W1[DM:]
    wpa = Wp[:DG, 0].reshape(1, DG)
    wpb = Wp[DG:, 0].reshape(1, DG)
    grid = (B // TB,)
    full = lambda i: (0, 0)
    return pl.pallas_call(
        _tc_mlp_body,
        grid=grid,
        in_specs=[
            pl.BlockSpec((TB, DG), lambda i: (i, 0)),
            pl.BlockSpec((TB, DG), lambda i: (i, 0)),
            pl.BlockSpec((TB, DM), lambda i: (i, 0)),
            pl.BlockSpec((TB, DM), lambda i: (i, 0)),
            pl.BlockSpec((DM, 256), full),
            pl.BlockSpec((DM, 256), full),
            pl.BlockSpec((1, 256), full),
            pl.BlockSpec((256, 128), full),
            pl.BlockSpec((1, 128), full),
            pl.BlockSpec((128, DG), full),
            pl.BlockSpec((1, DG), full),
            pl.BlockSpec((1, DG), full),
            pl.BlockSpec((1, DG), full),
            pl.BlockSpec((1, 1), full),
        ],
        out_specs=pl.BlockSpec((TB,), lambda i: (i,)),
        out_shape=jax.ShapeDtypeStruct((B,), f32),
    )(ug, ig, um, im, w1a, w1b, b1.reshape(1, 256), W2, b2.reshape(1, 128),
      W3, b3.reshape(1, DG), wpa, wpb, bp.reshape(1, 1))


def kernel(user, item, embed_user_GMF, embed_item_GMF, embed_user_MLP,
           embed_item_MLP, W1, b1, W2, b2, W3, b3, Wp, bp):
    ug, ig, um, im = _sc_gather(user, item, embed_user_GMF, embed_item_GMF,
                                embed_user_MLP, embed_item_MLP)
    return _tc_mlp(ug, ig, um, im, W1, b1, W2, b2, W3, b3, Wp, bp)


# speedup vs baseline: 1.9273x; 1.9273x over previous
import functools

import jax
import jax.numpy as jnp
from jax import lax
from jax.experimental import pallas as pl
from jax.experimental.pallas import tpu as pltpu
from jax.experimental.pallas import tpu_sc as plsc

NC, NS = 2, 16
NW = NC * NS
B = 16384
BW = B // NW
C = 128
NCHUNK = BW // C
DG = 64
DM = 256


def _sc_mlp_body(user_hbm, item_hbm, um_tab, im_tab, um_out, im_out,
                 idx_u, idx_i, um_v, im_v, s3, s4):
    wid = lax.axis_index("s") * NC + lax.axis_index("c")
    for c in range(NCHUNK):
        base = wid * BW + c * C
        pltpu.sync_copy(user_hbm.at[pl.ds(base, C)], idx_u)
        pltpu.sync_copy(item_hbm.at[pl.ds(base, C)], idx_i)
        cp_um = pltpu.async_copy(um_tab.at[idx_u], um_v, s3)
        cp_im = pltpu.async_copy(im_tab.at[idx_i], im_v, s4)
        cp_um.wait()
        pltpu.sync_copy(um_v, um_out.at[pl.ds(base, C)])
        cp_im.wait()
        pltpu.sync_copy(im_v, im_out.at[pl.ds(base, C)])


def _sc_gmf_body(user_hbm, item_hbm, ug_tab, ig_tab, ug_out, ig_out,
                 idx_u, idx_i, ug_v, ig_v, s1, s2):
    wid = lax.axis_index("s") * NC + lax.axis_index("c")
    for c in range(NCHUNK):
        base = wid * BW + c * C
        pltpu.sync_copy(user_hbm.at[pl.ds(base, C)], idx_u)
        pltpu.sync_copy(item_hbm.at[pl.ds(base, C)], idx_i)
        cp_ug = pltpu.async_copy(ug_tab.at[idx_u], ug_v, s1)
        cp_ig = pltpu.async_copy(ig_tab.at[idx_i], ig_v, s2)
        cp_ug.wait()
        pltpu.sync_copy(ug_v, ug_out.at[pl.ds(base, C)])
        cp_ig.wait()
        pltpu.sync_copy(ig_v, ig_out.at[pl.ds(base, C)])


def _mesh():
    return plsc.VectorSubcoreMesh(core_axis_name="c", subcore_axis_name="s",
                                  num_cores=NC, num_subcores=NS)


def _sc_gather_mlp(user, item, um_tab, im_tab):
    f32 = jnp.float32
    fn = pl.kernel(
        _sc_mlp_body,
        out_type=[
            jax.ShapeDtypeStruct((B, DM), f32),
            jax.ShapeDtypeStruct((B, DM), f32),
        ],
        mesh=_mesh(),
        scratch_types=[
            pltpu.VMEM((C,), jnp.int32),
            pltpu.VMEM((C,), jnp.int32),
            pltpu.VMEM((C, DM), f32),
            pltpu.VMEM((C, DM), f32),
            pltpu.SemaphoreType.DMA,
            pltpu.SemaphoreType.DMA,
        ],
    )
    return fn(user, item, um_tab, im_tab)


def _sc_gather_gmf(user, item, ug_tab, ig_tab):
    f32 = jnp.float32
    fn = pl.kernel(
        _sc_gmf_body,
        compiler_params=pltpu.CompilerParams(use_tc_tiling_on_sc=False),
        out_type=[
            jax.ShapeDtypeStruct((B, DG), f32),
            jax.ShapeDtypeStruct((B, DG), f32),
        ],
        mesh=_mesh(),
        scratch_types=[
            pltpu.VMEM((C,), jnp.int32),
            pltpu.VMEM((C,), jnp.int32),
            pltpu.VMEM((C, DG), f32),
            pltpu.VMEM((C, DG), f32),
            pltpu.SemaphoreType.DMA,
            pltpu.SemaphoreType.DMA,
        ],
    )
    return fn(user, item, ug_tab, ig_tab)


TB = 2048


def _tc_mlp_body(ug, ig, um, im, w1a, w1b, b1, w2, b2, w3, b3, wpa, wpb, bp,
                 out):
    f32 = jnp.float32
    hp = jax.lax.Precision.HIGHEST
    h = jnp.dot(um[...], w1a[...], precision=hp, preferred_element_type=f32)
    h += jnp.dot(im[...], w1b[...], precision=hp, preferred_element_type=f32)
    h = jnp.maximum(h + b1[...], 0.0)
    h = jnp.maximum(
        jnp.dot(h, w2[...], precision=hp, preferred_element_type=f32)
        + b2[...], 0.0)
    h = jnp.maximum(
        jnp.dot(h, w3[...], precision=hp, preferred_element_type=f32)
        + b3[...], 0.0)
    gmf = ug[...] * ig[...]
    pred = (jnp.sum(gmf * wpa[...], axis=1) + jnp.sum(h * wpb[...], axis=1)
            + bp[0, 0])
    out[...] = pred


def _tc_mlp(ug, ig, um, im, W1, b1, W2, b2, W3, b3, Wp, bp):
    f32 = jnp.float32
    w1a, w1b = W1[:DM], W1[DM:]
    wpa = Wp[:DG, 0].reshape(1, DG)
    wpb = Wp[DG:, 0].reshape(1, DG)
    grid = (B // TB,)
    full = lambda i: (0, 0)
    return pl.pallas_call(
        _tc_mlp_body,
        grid=grid,
        in_specs=[
            pl.BlockSpec((TB, DG), lambda i: (i, 0)),
            pl.BlockSpec((TB, DG), lambda i: (i, 0)),
            pl.BlockSpec((TB, DM), lambda i: (i, 0)),
            pl.BlockSpec((TB, DM), lambda i: (i, 0)),
            pl.BlockSpec((DM, 256), full),
            pl.BlockSpec((DM, 256), full),
            pl.BlockSpec((1, 256), full),
            pl.BlockSpec((256, 128), full),
            pl.BlockSpec((1, 128), full),
            pl.BlockSpec((128, DG), full),
            pl.BlockSpec((1, DG), full),
            pl.BlockSpec((1, DG), full),
            pl.BlockSpec((1, DG), full),
            pl.BlockSpec((1, 1), full),
        ],
        out_specs=pl.BlockSpec((TB,), lambda i: (i,)),
        out_shape=jax.ShapeDtypeStruct((B,), f32),
    )(ug, ig, um, im, w1a, w1b, b1.reshape(1, 256), W2, b2.reshape(1, 128),
      W3, b3.reshape(1, DG), wpa, wpb, bp.reshape(1, 1))


def kernel(user, item, embed_user_GMF, embed_item_GMF, embed_user_MLP,
           embed_item_MLP, W1, b1, W2, b2, W3, b3, Wp, bp):
    um, im = _sc_gather_mlp(user, item, embed_user_MLP, embed_item_MLP)
    ug, ig = _sc_gather_gmf(user, item, embed_user_GMF, embed_item_GMF)
    return _tc_mlp(ug, ig, um, im, W1, b1, W2, b2, W3, b3, Wp, bp)


# speedup vs baseline: 1.9300x; 1.0014x over previous
import functools

import jax
import jax.numpy as jnp
from jax import lax
from jax.experimental import pallas as pl
from jax.experimental.pallas import tpu as pltpu
from jax.experimental.pallas import tpu_sc as plsc

NC, NS = 2, 16
NW = NC * NS
B = 16384
BW = B // NW
C = 128
NCHUNK = BW // C
DG = 64
DP = 128
DM = 256


def _sc_gather_body(user_hbm, item_hbm, userh_hbm, itemh_hbm,
                    um_tab, im_tab, ug2_tab, ig2_tab,
                    um_out, im_out, ug_out, ig_out,
                    idx_u, idx_i, idx_uh, idx_ih,
                    um_v, im_v, ug_v, ig_v, s1, s2, s3, s4):
    wid = lax.axis_index("s") * NC + lax.axis_index("c")
    for c in range(NCHUNK):
        base = wid * BW + c * C
        pltpu.sync_copy(user_hbm.at[pl.ds(base, C)], idx_u)
        pltpu.sync_copy(item_hbm.at[pl.ds(base, C)], idx_i)
        pltpu.sync_copy(userh_hbm.at[pl.ds(base, C)], idx_uh)
        pltpu.sync_copy(itemh_hbm.at[pl.ds(base, C)], idx_ih)
        cp_um = pltpu.async_copy(um_tab.at[idx_u], um_v, s1)
        cp_im = pltpu.async_copy(im_tab.at[idx_i], im_v, s2)
        cp_ug = pltpu.async_copy(ug2_tab.at[idx_uh], ug_v, s3)
        cp_ig = pltpu.async_copy(ig2_tab.at[idx_ih], ig_v, s4)
        cp_um.wait()
        pltpu.sync_copy(um_v, um_out.at[pl.ds(base, C)])
        cp_im.wait()
        pltpu.sync_copy(im_v, im_out.at[pl.ds(base, C)])
        cp_ug.wait()
        pltpu.sync_copy(ug_v, ug_out.at[pl.ds(base, C)])
        cp_ig.wait()
        pltpu.sync_copy(ig_v, ig_out.at[pl.ds(base, C)])


def _sc_gather(user, item, userh, itemh, um_tab, im_tab, ug2_tab, ig2_tab):
    f32 = jnp.float32
    mesh = plsc.VectorSubcoreMesh(core_axis_name="c", subcore_axis_name="s",
                                  num_cores=NC, num_subcores=NS)
    fn = pl.kernel(
        _sc_gather_body,
        out_type=[
            jax.ShapeDtypeStruct((B, DM), f32),
            jax.ShapeDtypeStruct((B, DM), f32),
            jax.ShapeDtypeStruct((B, DP), f32),
            jax.ShapeDtypeStruct((B, DP), f32),
        ],
        mesh=mesh,
        scratch_types=[
            pltpu.VMEM((C,), jnp.int32),
            pltpu.VMEM((C,), jnp.int32),
            pltpu.VMEM((C,), jnp.int32),
            pltpu.VMEM((C,), jnp.int32),
            pltpu.VMEM((C, DM), f32),
            pltpu.VMEM((C, DM), f32),
            pltpu.VMEM((C, DP), f32),
            pltpu.VMEM((C, DP), f32),
            pltpu.SemaphoreType.DMA,
            pltpu.SemaphoreType.DMA,
            pltpu.SemaphoreType.DMA,
            pltpu.SemaphoreType.DMA,
        ],
    )
    return fn(user, item, userh, itemh, um_tab, im_tab, ug2_tab, ig2_tab)


TB = 2048


def _tc_mlp_body(ugp, igp, pu, pi, um, im, w1a, w1b, b1, w2, b2, w3, b3,
                 wpa, wpb, bp, out):
    f32 = jnp.float32
    hp = jax.lax.Precision.HIGHEST
    h = jnp.dot(um[...], w1a[...], precision=hp, preferred_element_type=f32)
    h += jnp.dot(im[...], w1b[...], precision=hp, preferred_element_type=f32)
    h = jnp.maximum(h + b1[...], 0.0)
    h = jnp.maximum(
        jnp.dot(h, w2[...], precision=hp, preferred_element_type=f32)
        + b2[...], 0.0)
    h = jnp.maximum(
        jnp.dot(h, w3[...], precision=hp, preferred_element_type=f32)
        + b3[...], 0.0)
    ugr = ugp[...]
    igr = igp[...]
    ug = ugr[:, :DG] + (ugr[:, DG:] - ugr[:, :DG]) * pu[...]
    ig = igr[:, :DG] + (igr[:, DG:] - igr[:, :DG]) * pi[...]
    gmf = ug * ig
    pred = (jnp.sum(gmf * wpa[...], axis=1) + jnp.sum(h * wpb[...], axis=1)
            + bp[0, 0])
    out[...] = pred


def _tc_mlp(ugp, igp, pu, pi, um, im, W1, b1, W2, b2, W3, b3, Wp, bp):
    f32 = jnp.float32
    w1a, w1b = W1[:DM], W1[DM:]
    wpa = Wp[:DG, 0].reshape(1, DG)
    wpb = Wp[DG:, 0].reshape(1, DG)
    grid = (B // TB,)
    full = lambda i: (0, 0)
    return pl.pallas_call(
        _tc_mlp_body,
        grid=grid,
        in_specs=[
            pl.BlockSpec((TB, DP), lambda i: (i, 0)),
            pl.BlockSpec((TB, DP), lambda i: (i, 0)),
            pl.BlockSpec((TB, 1), lambda i: (i, 0)),
            pl.BlockSpec((TB, 1), lambda i: (i, 0)),
            pl.BlockSpec((TB, DM), lambda i: (i, 0)),
            pl.BlockSpec((TB, DM), lambda i: (i, 0)),
            pl.BlockSpec((DM, 256), full),
            pl.BlockSpec((DM, 256), full),
            pl.BlockSpec((1, 256), full),
            pl.BlockSpec((256, 128), full),
            pl.BlockSpec((1, 128), full),
            pl.BlockSpec((128, DG), full),
            pl.BlockSpec((1, DG), full),
            pl.BlockSpec((1, DG), full),
            pl.BlockSpec((1, DG), full),
            pl.BlockSpec((1, 1), full),
        ],
        out_specs=pl.BlockSpec((TB,), lambda i: (i,)),
        out_shape=jax.ShapeDtypeStruct((B,), f32),
    )(ugp, igp, pu, pi, um, im, w1a, w1b, b1.reshape(1, 256), W2,
      b2.reshape(1, 128), W3, b3.reshape(1, DG), wpa, wpb, bp.reshape(1, 1))


def kernel(user, item, embed_user_GMF, embed_item_GMF, embed_user_MLP,
           embed_item_MLP, W1, b1, W2, b2, W3, b3, Wp, bp):
    ug2 = embed_user_GMF.reshape(embed_user_GMF.shape[0] // 2, DP)
    ig2 = embed_item_GMF.reshape(embed_item_GMF.shape[0] // 2, DP)
    userh = jax.lax.shift_right_logical(user, 1)
    itemh = jax.lax.shift_right_logical(item, 1)
    pu = jax.lax.convert_element_type(
        jax.lax.bitwise_and(user, 1), jnp.float32).reshape(B, 1)
    pi = jax.lax.convert_element_type(
        jax.lax.bitwise_and(item, 1), jnp.float32).reshape(B, 1)
    um, im, ugp, igp = _sc_gather(user, item, userh, itemh,
                                  embed_user_MLP, embed_item_MLP, ug2, ig2)
    return _tc_mlp(ugp, igp, pu, pi, um, im, W1, b1, W2, b2, W3, b3, Wp, bp)


# speedup vs baseline: 2.4965x; 1.2935x over previous
import functools

import jax
import jax.numpy as jnp
from jax import lax
from jax.experimental import pallas as pl
from jax.experimental.pallas import tpu as pltpu
from jax.experimental.pallas import tpu_sc as plsc

NC, NS = 2, 16
NW = NC * NS
B = 16384
BW = B // NW
CM = 128
CG = 32
DG = 64
DM = 256
TR = 8


def _sc_mlp_body(user_hbm, item_hbm, um_tab, im_tab, um_out, im_out,
                 idx_u, idx_i, um_v, im_v, s1, s2):
    wid = lax.axis_index("s") * NC + lax.axis_index("c")
    for c in range(BW // CM):
        base = wid * BW + c * CM
        pltpu.sync_copy(user_hbm.at[pl.ds(base, CM)], idx_u)
        pltpu.sync_copy(item_hbm.at[pl.ds(base, CM)], idx_i)
        cp_um = pltpu.async_copy(um_tab.at[idx_u], um_v, s1)
        cp_im = pltpu.async_copy(im_tab.at[idx_i], im_v, s2)
        cp_um.wait()
        pltpu.sync_copy(um_v, um_out.at[pl.ds(base, CM)])
        cp_im.wait()
        pltpu.sync_copy(im_v, im_out.at[pl.ds(base, CM)])


def _sc_gmf_body(userb_hbm, itemb_hbm, ug_tab, ig_tab, ug_out, ig_out,
                 idx_u, idx_i, ug_v, ig_v, s1, s2):
    wid = lax.axis_index("s") * NC + lax.axis_index("c")

    def chunk(c, carry):
        base = pl.multiple_of(wid * BW + c * CG, CG)
        pltpu.sync_copy(userb_hbm.at[pl.ds(base, CG)], idx_u)
        pltpu.sync_copy(itemb_hbm.at[pl.ds(base, CG)], idx_i)
        cps = []
        for g in range(CG // 16):
            ub = idx_u[pl.ds(g * 16, 16)] * TR
            ib = idx_i[pl.ds(g * 16, 16)] * TR
            for j in range(16):
                i = g * 16 + j
                cps.append(pltpu.async_copy(
                    ug_tab.at[pl.ds(pl.multiple_of(ub[j], TR), TR)],
                    ug_v.at[i], s1))
                cps.append(pltpu.async_copy(
                    ig_tab.at[pl.ds(pl.multiple_of(ib[j], TR), TR)],
                    ig_v.at[i], s2))
        for cp in cps:
            cp.wait()
        pltpu.sync_copy(ug_v, ug_out.at[pl.ds(base, CG)])
        pltpu.sync_copy(ig_v, ig_out.at[pl.ds(base, CG)])
        return carry

    lax.fori_loop(0, BW // CG, chunk, 0)


def _mesh():
    return plsc.VectorSubcoreMesh(core_axis_name="c", subcore_axis_name="s",
                                  num_cores=NC, num_subcores=NS)


def _sc_gather_mlp(user, item, um_tab, im_tab):
    f32 = jnp.float32
    fn = pl.kernel(
        _sc_mlp_body,
        out_type=[
            jax.ShapeDtypeStruct((B, DM), f32),
            jax.ShapeDtypeStruct((B, DM), f32),
        ],
        mesh=_mesh(),
        scratch_types=[
            pltpu.VMEM((CM,), jnp.int32),
            pltpu.VMEM((CM,), jnp.int32),
            pltpu.VMEM((CM, DM), f32),
            pltpu.VMEM((CM, DM), f32),
            pltpu.SemaphoreType.DMA,
            pltpu.SemaphoreType.DMA,
        ],
    )
    return fn(user, item, um_tab, im_tab)


def _sc_gather_gmf(userb, itemb, ug_tab, ig_tab):
    f32 = jnp.float32
    fn = pl.kernel(
        _sc_gmf_body,
        out_type=[
            jax.ShapeDtypeStruct((B, TR, DG), f32),
            jax.ShapeDtypeStruct((B, TR, DG), f32),
        ],
        mesh=_mesh(),
        scratch_types=[
            pltpu.VMEM((CG,), jnp.int32),
            pltpu.VMEM((CG,), jnp.int32),
            pltpu.VMEM((CG, TR, DG), f32),
            pltpu.VMEM((CG, TR, DG), f32),
            pltpu.SemaphoreType.DMA,
            pltpu.SemaphoreType.DMA,
        ],
    )
    return fn(userb, itemb, ug_tab, ig_tab)


TB = 1024


def _tc_mlp_body(ugt, igt, u7, i7, um, im, w1a, w1b, b1, w2, b2, w3, b3,
                 wpa, wpb, bp, out):
    f32 = jnp.float32
    hp = jax.lax.Precision.HIGHEST
    h = jnp.dot(um[...], w1a[...], precision=hp, preferred_element_type=f32)
    h += jnp.dot(im[...], w1b[...], precision=hp, preferred_element_type=f32)
    h = jnp.maximum(h + b1[...], 0.0)
    h = jnp.maximum(
        jnp.dot(h, w2[...], precision=hp, preferred_element_type=f32)
        + b2[...], 0.0)
    h = jnp.maximum(
        jnp.dot(h, w3[...], precision=hp, preferred_element_type=f32)
        + b3[...], 0.0)
    rows = lax.broadcasted_iota(jnp.int32, (TB, TR), 1).astype(f32)
    sel_u = jnp.where(rows == u7[...], 1.0, 0.0)[:, :, None]
    sel_i = jnp.where(rows == i7[...], 1.0, 0.0)[:, :, None]
    ug = jnp.sum(ugt[...] * sel_u, axis=1)
    ig = jnp.sum(igt[...] * sel_i, axis=1)
    gmf = ug * ig
    pred = (jnp.sum(gmf * wpa[...], axis=1) + jnp.sum(h * wpb[...], axis=1)
            + bp[0, 0])
    out[...] = pred


def _tc_mlp(ugt, igt, u7, i7, um, im, W1, b1, W2, b2, W3, b3, Wp, bp):
    f32 = jnp.float32
    w1a, w1b = W1[:DM], W1[DM:]
    wpa = Wp[:DG, 0].reshape(1, DG)
    wpb = Wp[DG:, 0].reshape(1, DG)
    grid = (B // TB,)
    full = lambda i: (0, 0)
    return pl.pallas_call(
        _tc_mlp_body,
        grid=grid,
        in_specs=[
            pl.BlockSpec((TB, TR, DG), lambda i: (i, 0, 0)),
            pl.BlockSpec((TB, TR, DG), lambda i: (i, 0, 0)),
            pl.BlockSpec((TB, 1), lambda i: (i, 0)),
            pl.BlockSpec((TB, 1), lambda i: (i, 0)),
            pl.BlockSpec((TB, DM), lambda i: (i, 0)),
            pl.BlockSpec((TB, DM), lambda i: (i, 0)),
            pl.BlockSpec((DM, 256), full),
            pl.BlockSpec((DM, 256), full),
            pl.BlockSpec((1, 256), full),
            pl.BlockSpec((256, 128), full),
            pl.BlockSpec((1, 128), full),
            pl.BlockSpec((128, DG), full),
            pl.BlockSpec((1, DG), full),
            pl.BlockSpec((1, DG), full),
            pl.BlockSpec((1, DG), full),
            pl.BlockSpec((1, 1), full),
        ],
        out_specs=pl.BlockSpec((TB,), lambda i: (i,)),
        out_shape=jax.ShapeDtypeStruct((B,), f32),
    )(ugt, igt, u7, i7, um, im, w1a, w1b, b1.reshape(1, 256), W2,
      b2.reshape(1, 128), W3, b3.reshape(1, DG), wpa, wpb, bp.reshape(1, 1))


def kernel(user, item, embed_user_GMF, embed_item_GMF, embed_user_MLP,
           embed_item_MLP, W1, b1, W2, b2, W3, b3, Wp, bp):
    f32 = jnp.float32
    userb = jax.lax.shift_right_logical(user, 3)
    itemb = jax.lax.shift_right_logical(item, 3)
    u7 = jax.lax.convert_element_type(
        jax.lax.bitwise_and(user, 7), f32).reshape(B, 1)
    i7 = jax.lax.convert_element_type(
        jax.lax.bitwise_and(item, 7), f32).reshape(B, 1)
    um, im = _sc_gather_mlp(user, item, embed_user_MLP, embed_item_MLP)
    ugt, igt = _sc_gather_gmf(userb, itemb, embed_user_GMF, embed_item_GMF)
    return _tc_mlp(ugt, igt, u7, i7, um, im, W1, b1, W2, b2, W3, b3, Wp, bp)
